# baseline (speedup 1.0000x reference)
import jax
import jax.numpy as jnp
from jax import lax
from jax.experimental import pallas as pl
from jax.experimental.pallas import tpu as pltpu, tpu_sc as plsc

D_MODEL = 1024
BATCH = 4
SC_BATCH = 3
SEQ = 4096

_info = plsc.get_sparse_core_info()
NC, NS, LANES = _info.num_cores, _info.num_subcores, _info.num_lanes
NW = NC * NS

CHUNK = 8
SEQ_PER_W = SEQ // NW
N_CHUNKS = SEQ_PER_W // CHUNK
DEPTH = 3
RING = DEPTH * SC_BATCH
VPG = 32
N_GROUPS = CHUNK * D_MODEL // (VPG * LANES)


def _body(x_hbm, pos_hbm, out_hbm, xb, posb, in_sems, out_sems, pos_sems):
    cid = lax.axis_index("c")
    sid = lax.axis_index("s")
    wid = sid * NC + cid
    seq0 = wid * SEQ_PER_W

    def sbase(ci):
        return lax.rem(ci, DEPTH) * SC_BATCH

    def in_copy(ci, q):
        s = sbase(ci) + q
        return pltpu.make_async_copy(
            x_hbm.at[q, pl.ds(seq0 + ci * CHUNK, CHUNK), :],
            xb.at[s], in_sems.at[s])

    def out_copy(ci, q):
        s = sbase(ci) + q
        return pltpu.make_async_copy(
            xb.at[s], out_hbm.at[q, pl.ds(seq0 + ci * CHUNK, CHUNK), :],
            out_sems.at[s])

    def pos_copy(ci):
        p = lax.rem(ci, 2)
        return pltpu.make_async_copy(
            pos_hbm.at[pl.ds(seq0 + ci * CHUNK, CHUNK), :],
            posb.at[p], pos_sems.at[p])

    pos_copy(0).start()
    for ci0 in range(min(DEPTH - 1, N_CHUNKS)):
        for q in range(SC_BATCH):
            in_copy(ci0, q).start()

    def chunk_step(ci, _):
        p = lax.rem(ci, 2)
        sb = sbase(ci)

        @pl.when(ci + 1 < N_CHUNKS)
        def _():
            pos_copy(ci + 1).start()

        for q in range(SC_BATCH):
            in_copy(ci, q).wait()
        pos_copy(ci).wait()

        def group(g, _):
            r = g >> 1
            coff = pl.multiple_of((g & 1) << 9, 512)
            vs = [posb[p, r, pl.ds(coff + u * LANES, LANES)]
                  for u in range(VPG)]
            for u in range(VPG):
                for q in range(SC_BATCH):
                    plsc.addupdate(
                        xb.at[sb + q, r, pl.ds(coff + u * LANES, LANES)],
                        vs[u])
            return 0

        lax.fori_loop(0, N_GROUPS, group, 0)

        @pl.when(ci >= 1)
        def _():
            for q in range(SC_BATCH):
                out_copy(ci - 1, q).wait()

        @pl.when(ci + DEPTH - 1 < N_CHUNKS)
        def _():
            for q in range(SC_BATCH):
                in_copy(ci + DEPTH - 1, q).start()

        for q in range(SC_BATCH):
            out_copy(ci, q).start()
        return 0

    lax.fori_loop(0, N_CHUNKS, chunk_step, 0)

    for q in range(SC_BATCH):
        out_copy(N_CHUNKS - 1, q).wait()


TC_BS = 256


def _tc_body(x_ref, pos_ref, o_ref):
    o_ref[0] = x_ref[0] + pos_ref[...]


@jax.jit
def kernel(x, pos_table):
    mesh = plsc.VectorSubcoreMesh(core_axis_name="c", subcore_axis_name="s")
    sc_out = pl.kernel(
        _body,
        out_type=jax.ShapeDtypeStruct((SC_BATCH, SEQ, D_MODEL), jnp.float32),
        mesh=mesh,
        scratch_types=[
            pltpu.VMEM((RING, CHUNK, D_MODEL), jnp.float32),
            pltpu.VMEM((2, CHUNK, D_MODEL), jnp.float32),
            pltpu.SemaphoreType.DMA((RING,)),
            pltpu.SemaphoreType.DMA((RING,)),
            pltpu.SemaphoreType.DMA((2,)),
        ],
        compiler_params=pltpu.CompilerParams(use_tc_tiling_on_sc=True),
    )(x, pos_table)

    tc_out = pl.pallas_call(
        _tc_body,
        grid=(SEQ // TC_BS,),
        in_specs=[
            pl.BlockSpec((1, TC_BS, D_MODEL), lambda i: (SC_BATCH, i, 0)),
            pl.BlockSpec((TC_BS, D_MODEL), lambda i: (i, 0)),
        ],
        out_specs=pl.BlockSpec((1, TC_BS, D_MODEL), lambda i: (0, i, 0)),
        out_shape=jax.ShapeDtypeStruct((1, SEQ, D_MODEL), jnp.float32),
    )(x, pos_table)

    sc_out, tc_out = lax.optimization_barrier((sc_out, tc_out))
    return lax.concatenate([sc_out, tc_out], dimension=0)


# speedup vs baseline: 1.5911x; 1.5911x over previous
import jax
import jax.numpy as jnp
from jax import lax
from jax.experimental import pallas as pl
from jax.experimental.pallas import tpu as pltpu, tpu_sc as plsc

D_MODEL = 1024
BATCH = 4
SEQ = 4096

_info = plsc.get_sparse_core_info()
NC, NS, LANES = _info.num_cores, _info.num_subcores, _info.num_lanes
NW = NC * NS

CHUNK = 8
SEQ_PER_W = SEQ // NW
N_CHUNKS = SEQ_PER_W // CHUNK
DEPTH = 3
RING = DEPTH * BATCH
VPG = 32
N_GROUPS = CHUNK * D_MODEL // (VPG * LANES)


def _body(x_hbm, pos_hbm, out_hbm, xb, posb, in_sems, out_sems, pos_sems):
    cid = lax.axis_index("c")
    sid = lax.axis_index("s")
    wid = sid * NC + cid
    seq0 = wid * SEQ_PER_W

    def sbase(ci):
        return lax.rem(ci, DEPTH) * BATCH

    def in_copy(ci, q):
        s = sbase(ci) + q
        return pltpu.make_async_copy(
            x_hbm.at[q, pl.ds(seq0 + ci * CHUNK, CHUNK), :],
            xb.at[s], in_sems.at[s])

    def out_copy(ci, q):
        s = sbase(ci) + q
        return pltpu.make_async_copy(
            xb.at[s], out_hbm.at[q, pl.ds(seq0 + ci * CHUNK, CHUNK), :],
            out_sems.at[s])

    def pos_copy(ci):
        p = lax.rem(ci, 2)
        return pltpu.make_async_copy(
            pos_hbm.at[pl.ds(seq0 + ci * CHUNK, CHUNK), :],
            posb.at[p], pos_sems.at[p])

    pos_copy(0).start()
    for ci0 in range(min(DEPTH - 1, N_CHUNKS)):
        for q in range(BATCH):
            in_copy(ci0, q).start()

    def chunk_step(ci, _):
        p = lax.rem(ci, 2)
        sb = sbase(ci)

        @pl.when(ci + 1 < N_CHUNKS)
        def _():
            pos_copy(ci + 1).start()

        for q in range(BATCH):
            in_copy(ci, q).wait()
        pos_copy(ci).wait()

        def group(g, _):
            r = g >> 1
            coff = pl.multiple_of((g & 1) << 9, 512)
            vs = [posb[p, r, pl.ds(coff + u * LANES, LANES)]
                  for u in range(VPG)]
            for u in range(VPG):
                for q in range(BATCH):
                    plsc.addupdate(
                        xb.at[sb + q, r, pl.ds(coff + u * LANES, LANES)],
                        vs[u])
            return 0

        lax.fori_loop(0, N_GROUPS, group, 0)

        @pl.when(ci >= 1)
        def _():
            for q in range(BATCH):
                out_copy(ci - 1, q).wait()

        @pl.when(ci + DEPTH - 1 < N_CHUNKS)
        def _():
            for q in range(BATCH):
                in_copy(ci + DEPTH - 1, q).start()

        for q in range(BATCH):
            out_copy(ci, q).start()
        return 0

    lax.fori_loop(0, N_CHUNKS, chunk_step, 0)

    for q in range(BATCH):
        out_copy(N_CHUNKS - 1, q).wait()


@jax.jit
def kernel(x, pos_table):
    mesh = plsc.VectorSubcoreMesh(core_axis_name="c", subcore_axis_name="s")
    return pl.kernel(
        _body,
        out_type=jax.ShapeDtypeStruct((BATCH, SEQ, D_MODEL), jnp.float32),
        mesh=mesh,
        scratch_types=[
            pltpu.VMEM((RING, CHUNK, D_MODEL), jnp.float32),
            pltpu.VMEM((2, CHUNK, D_MODEL), jnp.float32),
            pltpu.SemaphoreType.DMA((RING,)),
            pltpu.SemaphoreType.DMA((RING,)),
            pltpu.SemaphoreType.DMA((2,)),
        ],
        compiler_params=pltpu.CompilerParams(use_tc_tiling_on_sc=True),
    )(x, pos_table)
